# baseline (speedup 1.0000x reference)
import jax
import jax.numpy as jnp
from jax.experimental import pallas as pl
from jax.experimental.pallas import tpu as pltpu

_N = 100000
_D = 128
_BLOCK_ROWS = 20000


def _arc_loss_kernel(last_ref, a_ref, b_ref, out_ref):
    i = pl.program_id(0)

    t = a_ref[...] * b_ref[...]
    s = jnp.sum(t * t)

    @pl.when(i == 0)
    def _init():
        out_ref[...] = jnp.zeros_like(out_ref)

    out_ref[...] = out_ref[...] + s

    @pl.when(i == pl.num_programs(0) - 1)
    def _finish():
        denom = (last_ref[0] + 1).astype(jnp.float32)
        out_ref[...] = out_ref[...] / denom


def kernel(dx_dt, d2x_dt2, batch):
    num_blocks = _N // _BLOCK_ROWS
    last = batch[-1:].astype(jnp.int32)

    grid_spec = pltpu.PrefetchScalarGridSpec(
        num_scalar_prefetch=1,
        grid=(num_blocks,),
        in_specs=[
            pl.BlockSpec((_BLOCK_ROWS, _D), lambda i, s: (i, 0)),
            pl.BlockSpec((_BLOCK_ROWS, _D), lambda i, s: (i, 0)),
        ],
        out_specs=pl.BlockSpec((1, 1), lambda i, s: (0, 0)),
    )

    out = pl.pallas_call(
        _arc_loss_kernel,
        grid_spec=grid_spec,
        out_shape=jax.ShapeDtypeStruct((1, 1), jnp.float32),
        compiler_params=pltpu.CompilerParams(
            dimension_semantics=("arbitrary",),
        ),
    )(last, dx_dt, d2x_dt2)
    return out[0, 0]


# speedup vs baseline: 1.0400x; 1.0400x over previous
import jax
import jax.numpy as jnp
from jax.experimental import pallas as pl
from jax.experimental.pallas import tpu as pltpu

_N = 100000
_D = 128
_BLOCK_ROWS = 10000


def _arc_loss_kernel(last_ref, a_ref, b_ref, out_ref):
    i = pl.program_id(0)

    t = a_ref[...] * b_ref[...]
    s = jnp.sum(t * t)

    @pl.when(i == 0)
    def _init():
        out_ref[...] = jnp.zeros_like(out_ref)

    out_ref[...] = out_ref[...] + s

    @pl.when(i == pl.num_programs(0) - 1)
    def _finish():
        denom = (last_ref[0] + 1).astype(jnp.float32)
        out_ref[...] = out_ref[...] / denom


def kernel(dx_dt, d2x_dt2, batch):
    num_blocks = _N // _BLOCK_ROWS
    last = batch[-1:].astype(jnp.int32)

    grid_spec = pltpu.PrefetchScalarGridSpec(
        num_scalar_prefetch=1,
        grid=(num_blocks,),
        in_specs=[
            pl.BlockSpec((_BLOCK_ROWS, _D), lambda i, s: (i, 0)),
            pl.BlockSpec((_BLOCK_ROWS, _D), lambda i, s: (i, 0)),
        ],
        out_specs=pl.BlockSpec((1, 1), lambda i, s: (0, 0)),
    )

    out = pl.pallas_call(
        _arc_loss_kernel,
        grid_spec=grid_spec,
        out_shape=jax.ShapeDtypeStruct((1, 1), jnp.float32),
        compiler_params=pltpu.CompilerParams(
            dimension_semantics=("arbitrary",),
        ),
    )(last, dx_dt, d2x_dt2)
    return out[0, 0]
